# baseline (speedup 1.0000x reference)
import functools

import jax
import jax.numpy as jnp
from jax.experimental import pallas as pl
from jax.experimental.pallas import tpu as pltpu

_EPS = 1e-5


def _affine_cols(s_scr, g_ref, beta_ref, m):
    mean = s_scr[0:1, :] / m
    var = s_scr[1:2, :] / m - mean * mean
    inv = jax.lax.rsqrt(var + _EPS)
    a = g_ref[...] * inv
    c = beta_ref[...] - mean * a
    return a, c


def _accum_stats(y, s_scr, i):
    part = jnp.concatenate(
        [jnp.sum(y, axis=0, keepdims=True), jnp.sum(y * y, axis=0, keepdims=True)],
        axis=0,
    )

    @pl.when(i == 0)
    def _():
        s_scr[...] = part

    @pl.when(i != 0)
    def _():
        s_scr[...] += part


def _fused_body(
    x_ref, W1_ref, b1_ref, g1_ref, beta1_ref, W2_ref, b2_ref, g2_ref, beta2_ref,
    W3_ref, b3_ref, g3_ref, beta3_ref, Wl_ref, bl_ref,
    out_ref,
    y1_scr, y2_scr, y3_scr, acc_scr, s1_scr, s2_scr, s3_scr,
    Wc1_scr, bp2_scr, Wc2_scr, bp3_scr, Wp3l_scr, bpl_scr,
    *, m, h, co, r,
):
    s = pl.program_id(0)
    i = pl.program_id(1)
    rows = pl.ds(i * r, r)

    @pl.when(s == 0)
    def _():
        xb = x_ref[...].astype(jnp.bfloat16)
        y = jax.nn.relu(
            jnp.dot(xb, W1_ref[...], preferred_element_type=jnp.float32)
            + b1_ref[...]
        )
        y1_scr[rows, :] = y.astype(jnp.bfloat16)
        _accum_stats(y, s1_scr, i)

    @pl.when((s == 1) & (i == 0))
    def _():
        a, c = _affine_cols(s1_scr, g1_ref, beta1_ref, m)
        av = a.reshape(-1, 1)
        Wc1_scr[:, 0:h] = (W2_ref[...] * av).astype(jnp.bfloat16)
        Wc1_scr[:, h:] = (Wl_ref[0:h, :] * av).astype(jnp.bfloat16)
        bp2_scr[...] = (
            jnp.dot(c, W2_ref[...], preferred_element_type=jnp.float32)
            + b2_ref[...]
        )

    @pl.when(s == 1)
    def _():
        z = jnp.dot(
            y1_scr[rows, :], Wc1_scr[...], preferred_element_type=jnp.float32
        )
        y = jax.nn.relu(z[:, 0:h] + bp2_scr[...])
        y2_scr[rows, :] = y.astype(jnp.bfloat16)
        acc_scr[rows, :] = z[:, h:].astype(jnp.bfloat16)
        _accum_stats(y, s2_scr, i)

    @pl.when((s == 2) & (i == 0))
    def _():
        a, c = _affine_cols(s2_scr, g2_ref, beta2_ref, m)
        av = a.reshape(-1, 1)
        Wc2_scr[:, 0:co] = (W3_ref[...] * av).astype(jnp.bfloat16)
        Wc2_scr[:, co:] = (Wl_ref[h : 2 * h, :] * av).astype(jnp.bfloat16)
        bp3_scr[...] = (
            jnp.dot(c, W3_ref[...], preferred_element_type=jnp.float32)
            + b3_ref[...]
        )

    prows = pl.ds(i * (r // 2), r // 2)

    @pl.when(s == 2)
    def _():
        z = jnp.dot(
            y2_scr[rows, :], Wc2_scr[...], preferred_element_type=jnp.float32
        )
        y = jax.nn.relu(z[:, 0:co] + bp3_scr[...])
        y3_scr[prows, :] = y.astype(jnp.bfloat16).reshape(r // 2, 2 * co)
        acc_scr[rows, :] = (
            acc_scr[rows, :].astype(jnp.float32) + z[:, co:]
        ).astype(jnp.bfloat16)
        _accum_stats(y, s3_scr, i)

    @pl.when((s == 3) & (i == 0))
    def _():
        a1, c1 = _affine_cols(s1_scr, g1_ref, beta1_ref, m)
        a2, c2 = _affine_cols(s2_scr, g2_ref, beta2_ref, m)
        a3, c3 = _affine_cols(s3_scr, g3_ref, beta3_ref, m)
        Wp3l = (Wl_ref[2 * h :, :] * a3.reshape(-1, 1)).astype(jnp.bfloat16)
        zero = jnp.zeros((co, co), jnp.bfloat16)
        Wp3l_scr[0:co, 0:co] = Wp3l
        Wp3l_scr[0:co, co:] = zero
        Wp3l_scr[co:, 0:co] = zero
        Wp3l_scr[co:, co:] = Wp3l
        bpl_scr[...] = (
            jnp.dot(c1, Wl_ref[0:h, :], preferred_element_type=jnp.float32)
            + jnp.dot(c2, Wl_ref[h : 2 * h, :], preferred_element_type=jnp.float32)
            + jnp.dot(c3, Wl_ref[2 * h :, :], preferred_element_type=jnp.float32)
            + bl_ref[...]
        )

    @pl.when(s == 3)
    def _():
        zp = jnp.dot(
            y3_scr[prows, :], Wp3l_scr[...], preferred_element_type=jnp.float32
        )
        out_ref[...] = jax.nn.relu(
            acc_scr[rows, :].astype(jnp.float32) + zp.reshape(r, co) + bpl_scr[...]
        )


def kernel(x, adj, W1, b1, g1, beta1, W2, b2, g2, beta2, W3, b3, g3, beta3, Wl, bl):
    del adj
    B, N, Cin = x.shape
    H = W1.shape[1]
    Cout = W3.shape[1]
    M = B * N
    R = 10240
    nb = M // R
    mf = float(M)

    xf = x.reshape(M, Cin)
    row = lambda v: v.reshape(1, -1)
    full = lambda shape: pl.BlockSpec(shape, lambda s, i: (0, 0))

    out = pl.pallas_call(
        functools.partial(_fused_body, m=mf, h=H, co=Cout, r=R),
        grid=(4, nb),
        in_specs=[
            pl.BlockSpec((R, Cin), lambda s, i: (jnp.where(s == 0, i, 0), 0)),
            full((Cin, H)), full((1, H)), full((1, H)), full((1, H)),
            full((H, H)), full((1, H)), full((1, H)), full((1, H)),
            full((H, Cout)), full((1, Cout)), full((1, Cout)), full((1, Cout)),
            full((2 * H + Cout, Cout)), full((1, Cout)),
        ],
        out_specs=pl.BlockSpec((R, Cout), lambda s, i: (jnp.where(s == 3, i, 0), 0)),
        out_shape=jax.ShapeDtypeStruct((M, Cout), jnp.float32),
        scratch_shapes=[
            pltpu.VMEM((M, H), jnp.bfloat16),
            pltpu.VMEM((M, H), jnp.bfloat16),
            pltpu.VMEM((M // 2, 2 * Cout), jnp.bfloat16),
            pltpu.VMEM((M, Cout), jnp.bfloat16),
            pltpu.VMEM((2, H), jnp.float32),
            pltpu.VMEM((2, H), jnp.float32),
            pltpu.VMEM((2, Cout), jnp.float32),
            pltpu.VMEM((H, H + Cout), jnp.bfloat16),
            pltpu.VMEM((1, H), jnp.float32),
            pltpu.VMEM((H, 2 * Cout), jnp.bfloat16),
            pltpu.VMEM((1, Cout), jnp.float32),
            pltpu.VMEM((2 * Cout, 2 * Cout), jnp.bfloat16),
            pltpu.VMEM((1, Cout), jnp.float32),
        ],
    )(
        xf, W1.astype(jnp.bfloat16), row(b1), row(g1), row(beta1),
        W2, row(b2), row(g2), row(beta2),
        W3, row(b3), row(g3), row(beta3),
        Wl, row(bl),
    )

    return out.reshape(B, N, Cout)


# speedup vs baseline: 1.0104x; 1.0104x over previous
import functools

import jax
import jax.numpy as jnp
from jax.experimental import pallas as pl
from jax.experimental.pallas import tpu as pltpu

_EPS = 1e-5


def _affine_cols(s_scr, g_ref, beta_ref, m):
    mean = s_scr[0:1, :] / m
    var = s_scr[1:2, :] / m - mean * mean
    inv = jax.lax.rsqrt(var + _EPS)
    a = g_ref[...] * inv
    c = beta_ref[...] - mean * a
    return a, c


def _accum_stats(y, s_scr, i):
    part = jnp.concatenate(
        [jnp.sum(y, axis=0, keepdims=True), jnp.sum(y * y, axis=0, keepdims=True)],
        axis=0,
    )

    @pl.when(i == 0)
    def _():
        s_scr[...] = part

    @pl.when(i != 0)
    def _():
        s_scr[...] += part


def _fused_body(
    x_ref, W1_ref, b1_ref, g1_ref, beta1_ref, W2_ref, b2_ref, g2_ref, beta2_ref,
    W3_ref, b3_ref, g3_ref, beta3_ref, Wl_ref, bl_ref,
    out_ref,
    y1_scr, y2_scr, y3_scr, acc_scr, s1_scr, s2_scr, s3_scr,
    Wc1_scr, bp2_scr, Wc2_scr, bp3_scr, Wp3l_scr, bpl_scr,
    *, m, h, co, r,
):
    s = pl.program_id(0)
    i = pl.program_id(1)
    rows = pl.ds(i * r, r)

    @pl.when(s == 0)
    def _():
        xb = x_ref[...].astype(jnp.bfloat16)
        y = jax.nn.relu(
            jnp.dot(xb, W1_ref[...], preferred_element_type=jnp.float32)
            + b1_ref[...]
        )
        y1_scr[rows, :] = y.astype(jnp.bfloat16)
        _accum_stats(y, s1_scr, i)

    @pl.when((s == 1) & (i == 0))
    def _():
        a, c = _affine_cols(s1_scr, g1_ref, beta1_ref, m)
        av = a.reshape(-1, 1)
        Wc1_scr[:, 0:h] = (W2_ref[...] * av).astype(jnp.bfloat16)
        Wc1_scr[:, h:] = (Wl_ref[0:h, :] * av).astype(jnp.bfloat16)
        bp2_scr[...] = (
            jnp.dot(c, W2_ref[...], preferred_element_type=jnp.float32)
            + b2_ref[...]
        )

    @pl.when(s == 1)
    def _():
        z = jnp.dot(
            y1_scr[rows, :], Wc1_scr[...], preferred_element_type=jnp.float32
        )
        y = jax.nn.relu(z[:, 0:h] + bp2_scr[...])
        y2_scr[rows, :] = y.astype(jnp.bfloat16)
        acc_scr[rows, :] = z[:, h:].astype(jnp.bfloat16)
        _accum_stats(y, s2_scr, i)

    @pl.when((s == 2) & (i == 0))
    def _():
        a, c = _affine_cols(s2_scr, g2_ref, beta2_ref, m)
        av = a.reshape(-1, 1)
        Wc2_scr[:, 0:co] = (W3_ref[...] * av).astype(jnp.bfloat16)
        Wc2_scr[:, co:] = (Wl_ref[h : 2 * h, :] * av).astype(jnp.bfloat16)
        bp3_scr[...] = (
            jnp.dot(c, W3_ref[...], preferred_element_type=jnp.float32)
            + b3_ref[...]
        )

    @pl.when(s == 2)
    def _():
        z = jnp.dot(
            y2_scr[rows, :], Wc2_scr[...], preferred_element_type=jnp.float32
        )
        y = jax.nn.relu(z[:, 0:co] + bp3_scr[...])
        y3_scr[rows, :] = y.astype(jnp.bfloat16)
        acc_scr[rows, :] = (
            acc_scr[rows, :].astype(jnp.float32) + z[:, co:]
        ).astype(jnp.bfloat16)
        _accum_stats(y, s3_scr, i)

    @pl.when((s == 3) & (i == 0))
    def _():
        a1, c1 = _affine_cols(s1_scr, g1_ref, beta1_ref, m)
        a2, c2 = _affine_cols(s2_scr, g2_ref, beta2_ref, m)
        a3, c3 = _affine_cols(s3_scr, g3_ref, beta3_ref, m)
        Wp3l_scr[...] = (Wl_ref[2 * h :, :] * a3.reshape(-1, 1)).astype(jnp.bfloat16)
        bpl_scr[...] = (
            jnp.dot(c1, Wl_ref[0:h, :], preferred_element_type=jnp.float32)
            + jnp.dot(c2, Wl_ref[h : 2 * h, :], preferred_element_type=jnp.float32)
            + jnp.dot(c3, Wl_ref[2 * h :, :], preferred_element_type=jnp.float32)
            + bl_ref[...]
        )

    @pl.when(s == 3)
    def _():
        z = jnp.dot(
            y3_scr[rows, :], Wp3l_scr[...], preferred_element_type=jnp.float32
        )
        out_ref[...] = jax.nn.relu(
            acc_scr[rows, :].astype(jnp.float32) + z + bpl_scr[...]
        )


def kernel(x, adj, W1, b1, g1, beta1, W2, b2, g2, beta2, W3, b3, g3, beta3, Wl, bl):
    del adj
    B, N, Cin = x.shape
    H = W1.shape[1]
    Cout = W3.shape[1]
    M = B * N
    R = 10240
    nb = M // R
    mf = float(M)

    xf = x.reshape(M, Cin)
    row = lambda v: v.reshape(1, -1)
    full = lambda shape: pl.BlockSpec(shape, lambda s, i: (0, 0))

    out = pl.pallas_call(
        functools.partial(_fused_body, m=mf, h=H, co=Cout, r=R),
        grid=(4, nb),
        in_specs=[
            pl.BlockSpec((R, Cin), lambda s, i: (jnp.where(s == 0, i, 0), 0)),
            full((Cin, H)), full((1, H)), full((1, H)), full((1, H)),
            full((H, H)), full((1, H)), full((1, H)), full((1, H)),
            full((H, Cout)), full((1, Cout)), full((1, Cout)), full((1, Cout)),
            full((2 * H + Cout, Cout)), full((1, Cout)),
        ],
        out_specs=pl.BlockSpec((R, Cout), lambda s, i: (jnp.where(s == 3, i, 0), 0)),
        out_shape=jax.ShapeDtypeStruct((M, Cout), jnp.float32),
        scratch_shapes=[
            pltpu.VMEM((M, H), jnp.bfloat16),
            pltpu.VMEM((M, H), jnp.bfloat16),
            pltpu.VMEM((M, Cout), jnp.bfloat16),
            pltpu.VMEM((M, Cout), jnp.bfloat16),
            pltpu.VMEM((2, H), jnp.float32),
            pltpu.VMEM((2, H), jnp.float32),
            pltpu.VMEM((2, Cout), jnp.float32),
            pltpu.VMEM((H, H + Cout), jnp.bfloat16),
            pltpu.VMEM((1, H), jnp.float32),
            pltpu.VMEM((H, 2 * Cout), jnp.bfloat16),
            pltpu.VMEM((1, Cout), jnp.float32),
            pltpu.VMEM((Cout, Cout), jnp.bfloat16),
            pltpu.VMEM((1, Cout), jnp.float32),
        ],
    )(
        xf, W1.astype(jnp.bfloat16), row(b1), row(g1), row(beta1),
        W2, row(b2), row(g2), row(beta2),
        W3, row(b3), row(g3), row(beta3),
        Wl, row(bl),
    )

    return out.reshape(B, N, Cout)
